# Initial kernel scaffold; baseline (speedup 1.0000x reference)
import jax
import jax.numpy as jnp
from jax.experimental import pallas as pl


def kernel(x, pos, batch, x_skip, pos_skip, batch_skip, W, b):
    raise NotImplementedError("write your pallas kernel here")


# speedup vs baseline: 11.1063x; 11.1063x over previous
import functools

import jax
import jax.numpy as jnp
from jax import lax
from jax.experimental import pallas as pl
from jax.experimental.pallas import tpu as pltpu
from jax.experimental.pallas import tpu_sc as plsc

_N = 4096
_M = 16384
_DIN = 256
_DSKIP = 128
_DOUT = 256
_K = 3


_TM = 256


def _knn_body(ps_ref, posT_ref, idx_ref, w_ref):
    ps = ps_ref[...]
    posT = posT_ref[...]
    qq = jnp.sum(ps * ps, axis=1, keepdims=True)
    kk = jnp.sum(posT * posT, axis=0, keepdims=True)
    cross = jnp.dot(ps, posT, preferred_element_type=jnp.float32)
    d2 = qq + kk - 2.0 * cross

    iota = lax.broadcasted_iota(jnp.int32, d2.shape, 1)
    big = jnp.float32(3.4e38)
    vals, idxs = [], []
    cur = d2
    for _ in range(_K):
        m = jnp.min(cur, axis=1, keepdims=True)
        i = jnp.min(jnp.where(cur <= m, iota, _N), axis=1, keepdims=True)
        vals.append(m)
        idxs.append(i)
        cur = jnp.where(iota == i, big, cur)

    wk = [1.0 / jnp.maximum(jnp.maximum(v, 0.0), 1e-16) for v in vals]
    wsum = wk[0] + wk[1] + wk[2]
    idx_ref[...] = jnp.concatenate(idxs, axis=1)
    w_ref[...] = jnp.concatenate([w_ / wsum for w_ in wk], axis=1)


def _knn(pos_skip, posT):
    return pl.pallas_call(
        _knn_body,
        grid=(_M // _TM,),
        in_specs=[
            pl.BlockSpec((_TM, 3), lambda i: (i, 0)),
            pl.BlockSpec((3, _N), lambda i: (0, 0)),
        ],
        out_specs=[
            pl.BlockSpec((_TM, _K), lambda i: (i, 0)),
            pl.BlockSpec((_TM, _K), lambda i: (i, 0)),
        ],
        out_shape=[
            jax.ShapeDtypeStruct((_M, _K), jnp.int32),
            jax.ShapeDtypeStruct((_M, _K), jnp.float32),
        ],
    )(pos_skip, posT)


_NW = 32
_QW = _M // _NW
_CH = 32
_NCH = _QW // _CH


def _gather_body(x_hbm, idx_hbm, w_hbm, xi_hbm, idx_v, w_v, rows_v, out_v,
                 sem):
    wid = lax.axis_index("s") * 2 + lax.axis_index("c")

    def chunk_body(c, carry):
        base = wid * _QW + c * _CH
        b3 = base * 3
        pltpu.sync_copy(idx_hbm.at[pl.ds(b3, _CH * 3)], idx_v)
        pltpu.sync_copy(w_hbm.at[pl.ds(b3, _CH * 3)], w_v)
        pltpu.async_copy(x_hbm.at[idx_v], rows_v, sem).wait()

        def q_body(q, carry2):
            w0 = w_v[3 * q]
            w1 = w_v[3 * q + 1]
            w2 = w_v[3 * q + 2]
            for d in range(_DIN // 16):
                sl = pl.ds(16 * d, 16)
                out_v[q, sl] = (w0 * rows_v[3 * q, sl]
                                + w1 * rows_v[3 * q + 1, sl]
                                + w2 * rows_v[3 * q + 2, sl])
            return carry2

        lax.fori_loop(0, _CH, q_body, 0)
        pltpu.sync_copy(out_v, xi_hbm.at[pl.ds(base, _CH)])
        return carry

    lax.fori_loop(0, _NCH, chunk_body, 0)


@functools.lru_cache(maxsize=1)
def _make_gather():
    @functools.partial(
        pl.kernel,
        mesh=plsc.VectorSubcoreMesh(core_axis_name="c", subcore_axis_name="s"),
        out_type=jax.ShapeDtypeStruct((_M, _DIN), jnp.float32),
        scratch_types=[
            pltpu.VMEM((_CH * 3,), jnp.int32),
            pltpu.VMEM((_CH * 3, 16), jnp.float32),
            pltpu.VMEM((_CH * 3, _DIN), jnp.float32),
            pltpu.VMEM((_CH, _DIN), jnp.float32),
            pltpu.SemaphoreType.DMA,
        ],
    )
    def _gather(x_hbm, idx_hbm, w_hbm, xi_hbm, idx_v, w_v, rows_v, out_v,
                sem):
        _gather_body(x_hbm, idx_hbm, w_hbm, xi_hbm, idx_v, w_v, rows_v,
                     out_v, sem)

    return _gather


_TMC = 1024


def _mlp_body(xi_ref, xs_ref, w1_ref, w2_ref, b_ref, o_ref):
    h = jnp.dot(xi_ref[...], w1_ref[...], preferred_element_type=jnp.float32)
    h = h + jnp.dot(xs_ref[...], w2_ref[...], preferred_element_type=jnp.float32)
    o_ref[...] = jnp.maximum(h + b_ref[...], 0.0)


def _mlp(xi, x_skip, W1, W2, b2d):
    return pl.pallas_call(
        _mlp_body,
        grid=(_M // _TMC,),
        in_specs=[
            pl.BlockSpec((_TMC, _DIN), lambda i: (i, 0)),
            pl.BlockSpec((_TMC, _DSKIP), lambda i: (i, 0)),
            pl.BlockSpec((_DIN, _DOUT), lambda i: (0, 0)),
            pl.BlockSpec((_DSKIP, _DOUT), lambda i: (0, 0)),
            pl.BlockSpec((1, _DOUT), lambda i: (0, 0)),
        ],
        out_specs=pl.BlockSpec((_TMC, _DOUT), lambda i: (i, 0)),
        out_shape=jax.ShapeDtypeStruct((_M, _DOUT), jnp.float32),
    )(xi, x_skip, W1, W2, b2d)


def kernel(x, pos, batch, x_skip, pos_skip, batch_skip, W, b):
    posT = pos.T
    idx, w = _knn(pos_skip, posT)
    wexp = jnp.broadcast_to(w.reshape(_M * _K, 1), (_M * _K, 16))
    xi = _make_gather()(x, idx.reshape(-1), wexp)
    out = _mlp(xi, x_skip, W[:_DIN], W[_DIN:], b.reshape(1, _DOUT))
    return (out, pos_skip, batch_skip)


# speedup vs baseline: 12.7168x; 1.1450x over previous
import functools

import jax
import jax.numpy as jnp
from jax import lax
from jax.experimental import pallas as pl
from jax.experimental.pallas import tpu as pltpu
from jax.experimental.pallas import tpu_sc as plsc

_N = 4096
_M = 16384
_DIN = 256
_DSKIP = 128
_DOUT = 256
_K = 3


_TM = 256


def _knn_body(ps_ref, posT_ref, idx_ref, w_ref):
    ps = ps_ref[...]
    posT = posT_ref[...]
    qq = jnp.sum(ps * ps, axis=1, keepdims=True)
    kk = jnp.sum(posT * posT, axis=0, keepdims=True)
    cross = jnp.dot(ps, posT, preferred_element_type=jnp.float32)
    d2 = qq + kk - 2.0 * cross

    iota = lax.broadcasted_iota(jnp.int32, d2.shape, 1)
    big = jnp.float32(3.4e38)
    vals, idxs = [], []
    cur = d2
    for _ in range(_K):
        m = jnp.min(cur, axis=1, keepdims=True)
        i = jnp.min(jnp.where(cur <= m, iota, _N), axis=1, keepdims=True)
        vals.append(m)
        idxs.append(i)
        cur = jnp.where(iota == i, big, cur)

    wk = [1.0 / jnp.maximum(jnp.maximum(v, 0.0), 1e-16) for v in vals]
    wsum = wk[0] + wk[1] + wk[2]
    idx_ref[...] = jnp.concatenate(idxs, axis=1)
    w_ref[...] = jnp.concatenate([w_ / wsum for w_ in wk], axis=1)


def _knn(pos_skip, posT):
    m = pos_skip.shape[0]
    return pl.pallas_call(
        _knn_body,
        grid=(m // _TM,),
        in_specs=[
            pl.BlockSpec((_TM, 3), lambda i: (i, 0)),
            pl.BlockSpec((3, _N), lambda i: (0, 0)),
        ],
        out_specs=[
            pl.BlockSpec((_TM, _K), lambda i: (i, 0)),
            pl.BlockSpec((_TM, _K), lambda i: (i, 0)),
        ],
        out_shape=[
            jax.ShapeDtypeStruct((m, _K), jnp.int32),
            jax.ShapeDtypeStruct((m, _K), jnp.float32),
        ],
    )(pos_skip, posT)


_NW = 32
_CH = 32
_NBUF = 2


def _make_gather_body(m_slab):
    qw = m_slab // _NW
    nch = qw // _CH

    def body(x_hbm, idx_hbm, w_hbm, xi_hbm, idx_v, w_v, rows_v, out_v,
             gsems, osems):
        wid = lax.axis_index("s") * 2 + lax.axis_index("c")
        q0 = wid * qw

        def out_copy(c, b):
            return pltpu.make_async_copy(
                out_v.at[b], xi_hbm.at[pl.ds(q0 + c * _CH, _CH)], osems[b])

        def start_chunk(c, b):
            base3 = (q0 + c * _CH) * 3
            pltpu.sync_copy(idx_hbm.at[pl.ds(base3, _CH * 3)], idx_v.at[b])
            pltpu.sync_copy(w_hbm.at[pl.ds(base3, _CH * 3)], w_v.at[b])
            pltpu.make_async_copy(x_hbm.at[idx_v.at[b]], rows_v.at[b],
                                  gsems[b]).start()

        def compute_chunk(c, b):
            pltpu.make_async_copy(x_hbm.at[idx_v.at[b]], rows_v.at[b],
                                  gsems[b]).wait()

            def q_body(q, carry2):
                w0 = w_v[b, 3 * q]
                w1 = w_v[b, 3 * q + 1]
                w2 = w_v[b, 3 * q + 2]
                for d in range(_DIN // 16):
                    sl = pl.ds(16 * d, 16)
                    out_v[b, q, sl] = (w0 * rows_v[b, 3 * q, sl]
                                       + w1 * rows_v[b, 3 * q + 1, sl]
                                       + w2 * rows_v[b, 3 * q + 2, sl])
                return carry2

            lax.fori_loop(0, _CH, q_body, 0)
            out_copy(c, b).start()

        for b in range(_NBUF):
            start_chunk(b, b)
        for c in range(nch):
            b = c % _NBUF
            if c >= _NBUF:
                out_copy(c - _NBUF, b).wait()
            compute_chunk(c, b)
            if c + _NBUF < nch:
                start_chunk(c + _NBUF, b)
        for c in range(nch - _NBUF, nch):
            out_copy(c, c % _NBUF).wait()

    return body


@functools.lru_cache(maxsize=2)
def _make_gather(m_slab):
    @functools.partial(
        pl.kernel,
        mesh=plsc.VectorSubcoreMesh(core_axis_name="c", subcore_axis_name="s"),
        out_type=jax.ShapeDtypeStruct((m_slab, _DIN), jnp.float32),
        scratch_types=[
            pltpu.VMEM((_NBUF, _CH * 3), jnp.int32),
            pltpu.VMEM((_NBUF, _CH * 3, 16), jnp.float32),
            pltpu.VMEM((_NBUF, _CH * 3, _DIN), jnp.float32),
            pltpu.VMEM((_NBUF, _CH, _DIN), jnp.float32),
            pltpu.SemaphoreType.DMA,
            pltpu.SemaphoreType.DMA,
            pltpu.SemaphoreType.DMA,
            pltpu.SemaphoreType.DMA,
        ],
    )
    def _gather(x_hbm, idx_hbm, w_hbm, xi_hbm, idx_v, w_v, rows_v, out_v,
                gsem0, gsem1, osem0, osem1):
        _make_gather_body(m_slab)(x_hbm, idx_hbm, w_hbm, xi_hbm, idx_v, w_v,
                                  rows_v, out_v, (gsem0, gsem1),
                                  (osem0, osem1))

    return _gather


_TMC = 1024


def _mlp_body(xi_ref, xs_ref, w1_ref, w2_ref, b_ref, o_ref):
    h = jnp.dot(xi_ref[...], w1_ref[...], preferred_element_type=jnp.float32)
    h = h + jnp.dot(xs_ref[...], w2_ref[...], preferred_element_type=jnp.float32)
    o_ref[...] = jnp.maximum(h + b_ref[...], 0.0)


def _mlp(xi, x_skip, W1, W2, b2d):
    m = xi.shape[0]
    return pl.pallas_call(
        _mlp_body,
        grid=(m // _TMC,),
        in_specs=[
            pl.BlockSpec((_TMC, _DIN), lambda i: (i, 0)),
            pl.BlockSpec((_TMC, _DSKIP), lambda i: (i, 0)),
            pl.BlockSpec((_DIN, _DOUT), lambda i: (0, 0)),
            pl.BlockSpec((_DSKIP, _DOUT), lambda i: (0, 0)),
            pl.BlockSpec((1, _DOUT), lambda i: (0, 0)),
        ],
        out_specs=pl.BlockSpec((_TMC, _DOUT), lambda i: (i, 0)),
        out_shape=jax.ShapeDtypeStruct((m, _DOUT), jnp.float32),
    )(xi, x_skip, W1, W2, b2d)


_NSLAB = 2


def kernel(x, pos, batch, x_skip, pos_skip, batch_skip, W, b):
    posT = pos.T
    W1, W2, b2d = W[:_DIN], W[_DIN:], b.reshape(1, _DOUT)
    ms = _M // _NSLAB
    outs = []
    for s in range(_NSLAB):
        sl = slice(s * ms, (s + 1) * ms)
        idx, w = _knn(pos_skip[sl], posT)
        wexp = jnp.broadcast_to(w.reshape(ms * _K, 1), (ms * _K, 16))
        xi = _make_gather(ms)(x, idx.reshape(-1), wexp)
        outs.append(_mlp(xi, x_skip[sl], W1, W2, b2d))
    out = jnp.concatenate(outs, axis=0)
    return (out, pos_skip, batch_skip)
